# baseline (speedup 1.0000x reference)
import functools
import math

import jax
import jax.numpy as jnp
from jax import lax
from jax.experimental import pallas as pl
from jax.experimental.pallas import tpu as pltpu
from jax.experimental.pallas import tpu_sc as plsc

HIDDEN = 1024
MEM = 512
BUCKETS = 100000
SLOTS = 8
SLOT_DIM = MEM // SLOTS
N = 16384

NC = 2
NS = 16
NW = NC * NS
TOTAL_ROWS = N * SLOTS
ROWS_PER_W = TOTAL_ROWS // NW
CHUNK = 128
CHUNKS_PER_W = ROWS_PER_W // CHUNK


def _sc_gather(table_hbm, idx_hbm, out_hbm, idx_v, rows_v, sem):
    wid = lax.axis_index("s") * NC + lax.axis_index("c")
    base = wid * ROWS_PER_W
    pltpu.sync_copy(idx_hbm.at[wid], idx_v)

    def body(j, carry):
        pltpu.async_copy(table_hbm.at[idx_v.at[j]], rows_v, sem).wait()
        pltpu.sync_copy(rows_v, out_hbm.at[pl.ds(base + j * CHUNK, CHUNK)])
        return carry

    lax.fori_loop(0, CHUNKS_PER_W, body, 0)


def _make_gather_call():
    return functools.partial(
        pl.kernel,
        out_type=jax.ShapeDtypeStruct((TOTAL_ROWS, SLOT_DIM), jnp.float32),
        mesh=plsc.VectorSubcoreMesh(core_axis_name="c", subcore_axis_name="s",
                                    num_cores=NC, num_subcores=NS),
        scratch_types=[
            pltpu.VMEM((CHUNKS_PER_W, CHUNK), jnp.int32),
            pltpu.VMEM((CHUNK, SLOT_DIM), jnp.float32),
            pltpu.SemaphoreType.DMA,
        ],
        compiler_params=pltpu.CompilerParams(use_tc_tiling_on_sc=False),
    )(_sc_gather)


def _dense_body(hid_ref, mem_ref, wkt_ref, wvt_ref, qn_ref, kn_ref, vn_ref,
                cw_ref, cb_ref, out_ref):
    eps = 1e-8
    q = hid_ref[...]
    q = q * lax.rsqrt(jnp.mean(q * q, axis=-1, keepdims=True) + eps)
    q = q * qn_ref[...]
    m = mem_ref[...]
    m_hi = m.astype(jnp.bfloat16)
    m_lo = (m - m_hi.astype(jnp.float32)).astype(jnp.bfloat16)

    def matmul(w):
        w_hi = w.astype(jnp.bfloat16)
        w_lo = (w - w_hi.astype(jnp.float32)).astype(jnp.bfloat16)
        acc = jnp.dot(m_hi, w_hi, preferred_element_type=jnp.float32)
        acc += jnp.dot(m_hi, w_lo, preferred_element_type=jnp.float32)
        acc += jnp.dot(m_lo, w_hi, preferred_element_type=jnp.float32)
        return acc

    k = matmul(wkt_ref[...])
    k = k * lax.rsqrt(jnp.mean(k * k, axis=-1, keepdims=True) + eps)
    k = k * kn_ref[...]
    v = matmul(wvt_ref[...])
    v = v * lax.rsqrt(jnp.mean(v * v, axis=-1, keepdims=True) + eps)
    v = v * vn_ref[...]
    logits = jnp.sum(q * k, axis=-1, keepdims=True) * (1.0 / math.sqrt(HIDDEN))
    alpha = jax.nn.sigmoid(logits)
    g = alpha * v
    co = g * cw_ref[...] + cb_ref[...]
    out_ref[...] = co * jax.nn.sigmoid(co) + g


def kernel(hidden, batch_ngram_bucket_ids, tables, Wk, Wv, qn_w, kn_w, vn_w,
           conv_w, conv_b):
    ids = jnp.asarray(batch_ngram_bucket_ids, jnp.int32)
    flat_ids = ids + (jnp.arange(SLOTS, dtype=jnp.int32) * BUCKETS)[None, :]
    idx = flat_ids.reshape(NW, CHUNKS_PER_W, CHUNK)
    flat_tables = tables.reshape(SLOTS * BUCKETS, SLOT_DIM)

    rows = _make_gather_call()(flat_tables, idx)
    memory = rows.reshape(N, MEM)

    bn = 1024
    grid = (N // bn,)
    full = lambda i: (0, 0)
    vec = lambda x: x.reshape(1, HIDDEN)
    out = pl.pallas_call(
        _dense_body,
        grid=grid,
        in_specs=[
            pl.BlockSpec((bn, HIDDEN), lambda i: (i, 0)),
            pl.BlockSpec((bn, MEM), lambda i: (i, 0)),
            pl.BlockSpec((MEM, HIDDEN), full),
            pl.BlockSpec((MEM, HIDDEN), full),
            pl.BlockSpec((1, HIDDEN), full),
            pl.BlockSpec((1, HIDDEN), full),
            pl.BlockSpec((1, HIDDEN), full),
            pl.BlockSpec((1, HIDDEN), full),
            pl.BlockSpec((1, HIDDEN), full),
        ],
        out_specs=pl.BlockSpec((bn, HIDDEN), lambda i: (i, 0)),
        out_shape=jax.ShapeDtypeStruct((N, HIDDEN), jnp.float32),
    )(hidden, memory, Wk.T, Wv.T, vec(qn_w), vec(kn_w), vec(vn_w),
      vec(conv_w[:, 0, 2]), vec(conv_b))
    return out


# speedup vs baseline: 1.1154x; 1.1154x over previous
import functools
import math

import jax
import jax.numpy as jnp
from jax import lax
from jax.experimental import pallas as pl
from jax.experimental.pallas import tpu as pltpu
from jax.experimental.pallas import tpu_sc as plsc

HIDDEN = 1024
MEM = 512
BUCKETS = 100000
SLOTS = 8
SLOT_DIM = MEM // SLOTS
N = 16384

NC = 2
NS = 16
NW = NC * NS
TOTAL_ROWS = N * SLOTS
ROWS_PER_W = TOTAL_ROWS // NW
CHUNK = 128
CHUNKS_PER_W = ROWS_PER_W // CHUNK


def _sc_gather(table_hbm, idx_hbm, out_hbm, idx_v, rows_v, sem):
    wid = lax.axis_index("s") * NC + lax.axis_index("c")
    base = wid * ROWS_PER_W
    pltpu.sync_copy(idx_hbm.at[wid], idx_v)

    def body(j, carry):
        pltpu.async_copy(table_hbm.at[idx_v.at[j]], rows_v, sem).wait()
        pltpu.sync_copy(rows_v, out_hbm.at[pl.ds(base + j * CHUNK, CHUNK)])
        return carry

    lax.fori_loop(0, CHUNKS_PER_W, body, 0)


def _make_gather_call():
    return functools.partial(
        pl.kernel,
        out_type=jax.ShapeDtypeStruct((TOTAL_ROWS, SLOT_DIM), jnp.float32),
        mesh=plsc.VectorSubcoreMesh(core_axis_name="c", subcore_axis_name="s",
                                    num_cores=NC, num_subcores=NS),
        scratch_types=[
            pltpu.VMEM((CHUNKS_PER_W, CHUNK), jnp.int32),
            pltpu.VMEM((CHUNK, SLOT_DIM), jnp.float32),
            pltpu.SemaphoreType.DMA,
        ],
        compiler_params=pltpu.CompilerParams(use_tc_tiling_on_sc=False),
    )(_sc_gather)


def _dense_body(hid_ref, mem_ref, wkt_ref, wvt_ref, qn_ref, kn_ref, vn_ref,
                cw_ref, cb_ref, out_ref):
    eps = 1e-8
    q = hid_ref[...]
    q = q * lax.rsqrt(jnp.mean(q * q, axis=-1, keepdims=True) + eps)
    q = q * qn_ref[...]
    m = mem_ref[...]
    m_hi = m.astype(jnp.bfloat16)

    def matmul(w):
        return jnp.dot(m_hi, w.astype(jnp.bfloat16),
                       preferred_element_type=jnp.float32)

    k = matmul(wkt_ref[...])
    k = k * lax.rsqrt(jnp.mean(k * k, axis=-1, keepdims=True) + eps)
    k = k * kn_ref[...]
    v = matmul(wvt_ref[...])
    v = v * lax.rsqrt(jnp.mean(v * v, axis=-1, keepdims=True) + eps)
    v = v * vn_ref[...]
    logits = jnp.sum(q * k, axis=-1, keepdims=True) * (1.0 / math.sqrt(HIDDEN))
    alpha = jax.nn.sigmoid(logits)
    g = alpha * v
    co = g * cw_ref[...] + cb_ref[...]
    out_ref[...] = co * jax.nn.sigmoid(co) + g


def kernel(hidden, batch_ngram_bucket_ids, tables, Wk, Wv, qn_w, kn_w, vn_w,
           conv_w, conv_b):
    ids = jnp.asarray(batch_ngram_bucket_ids, jnp.int32)
    flat_ids = ids + (jnp.arange(SLOTS, dtype=jnp.int32) * BUCKETS)[None, :]
    idx = flat_ids.reshape(NW, CHUNKS_PER_W, CHUNK)
    flat_tables = tables.reshape(SLOTS * BUCKETS, SLOT_DIM)

    rows = _make_gather_call()(flat_tables, idx)
    memory = rows.reshape(N, MEM)

    bn = 1024
    grid = (N // bn,)
    full = lambda i: (0, 0)
    vec = lambda x: x.reshape(1, HIDDEN)
    out = pl.pallas_call(
        _dense_body,
        grid=grid,
        in_specs=[
            pl.BlockSpec((bn, HIDDEN), lambda i: (i, 0)),
            pl.BlockSpec((bn, MEM), lambda i: (i, 0)),
            pl.BlockSpec((MEM, HIDDEN), full),
            pl.BlockSpec((MEM, HIDDEN), full),
            pl.BlockSpec((1, HIDDEN), full),
            pl.BlockSpec((1, HIDDEN), full),
            pl.BlockSpec((1, HIDDEN), full),
            pl.BlockSpec((1, HIDDEN), full),
            pl.BlockSpec((1, HIDDEN), full),
        ],
        out_specs=pl.BlockSpec((bn, HIDDEN), lambda i: (i, 0)),
        out_shape=jax.ShapeDtypeStruct((N, HIDDEN), jnp.float32),
    )(hidden, memory, Wk.T, Wv.T, vec(qn_w), vec(kn_w), vec(vn_w),
      vec(conv_w[:, 0, 2]), vec(conv_b))
    return out


# speedup vs baseline: 1.1225x; 1.0063x over previous
import functools
import math

import jax
import jax.numpy as jnp
from jax import lax
from jax.experimental import pallas as pl
from jax.experimental.pallas import tpu as pltpu
from jax.experimental.pallas import tpu_sc as plsc

HIDDEN = 1024
MEM = 512
BUCKETS = 100000
SLOTS = 8
SLOT_DIM = MEM // SLOTS
N = 16384

NC = 2
NS = 16
NW = NC * NS
ROWS_PER_W = N // NW
CHUNK = 128
CHUNKS_PER_SLOT = ROWS_PER_W // CHUNK


def _sc_gather(table_hbm, idx_hbm, out_hbm, idx_v, rows_v, sem):
    wid = lax.axis_index("s") * NC + lax.axis_index("c")
    n0 = wid * ROWS_PER_W
    pltpu.sync_copy(idx_hbm.at[:, pl.ds(n0, ROWS_PER_W)], idx_v)

    def body(j, carry):
        s = j // CHUNKS_PER_SLOT
        k = j % CHUNKS_PER_SLOT
        pltpu.async_copy(
            table_hbm.at[s].at[idx_v.at[s, pl.ds(k * CHUNK, CHUNK)]],
            rows_v, sem).wait()
        pltpu.sync_copy(
            rows_v,
            out_hbm.at[pl.ds(n0 + k * CHUNK, CHUNK),
                       pl.ds(s * SLOT_DIM, SLOT_DIM)])
        return carry

    lax.fori_loop(0, SLOTS * CHUNKS_PER_SLOT, body, 0)


def _make_gather_call():
    return functools.partial(
        pl.kernel,
        out_type=jax.ShapeDtypeStruct((N, MEM), jnp.float32),
        mesh=plsc.VectorSubcoreMesh(core_axis_name="c", subcore_axis_name="s",
                                    num_cores=NC, num_subcores=NS),
        scratch_types=[
            pltpu.VMEM((SLOTS, ROWS_PER_W), jnp.int32),
            pltpu.VMEM((CHUNK, SLOT_DIM), jnp.float32),
            pltpu.SemaphoreType.DMA,
        ],
        compiler_params=pltpu.CompilerParams(use_tc_tiling_on_sc=False),
    )(_sc_gather)


def _dense_body(hid_ref, mem_ref, wkt_ref, wvt_ref, qn_ref, kn_ref, vn_ref,
                cw_ref, cb_ref, out_ref):
    eps = 1e-8
    q = hid_ref[...]
    q = q * lax.rsqrt(jnp.mean(q * q, axis=-1, keepdims=True) + eps)
    q = q * qn_ref[...]
    m = mem_ref[...]
    m_hi = m.astype(jnp.bfloat16)

    def matmul(w):
        return jnp.dot(m_hi, w.astype(jnp.bfloat16),
                       preferred_element_type=jnp.float32)

    k = matmul(wkt_ref[...])
    k = k * lax.rsqrt(jnp.mean(k * k, axis=-1, keepdims=True) + eps)
    k = k * kn_ref[...]
    v = matmul(wvt_ref[...])
    v = v * lax.rsqrt(jnp.mean(v * v, axis=-1, keepdims=True) + eps)
    v = v * vn_ref[...]
    logits = jnp.sum(q * k, axis=-1, keepdims=True) * (1.0 / math.sqrt(HIDDEN))
    alpha = jax.nn.sigmoid(logits)
    g = alpha * v
    co = g * cw_ref[...] + cb_ref[...]
    out_ref[...] = co * jax.nn.sigmoid(co) + g


def kernel(hidden, batch_ngram_bucket_ids, tables, Wk, Wv, qn_w, kn_w, vn_w,
           conv_w, conv_b):
    idx = jnp.asarray(batch_ngram_bucket_ids, jnp.int32).T

    memory = _make_gather_call()(tables, idx)

    bn = 1024
    grid = (N // bn,)
    full = lambda i: (0, 0)
    vec = lambda x: x.reshape(1, HIDDEN)
    out = pl.pallas_call(
        _dense_body,
        grid=grid,
        in_specs=[
            pl.BlockSpec((bn, HIDDEN), lambda i: (i, 0)),
            pl.BlockSpec((bn, MEM), lambda i: (i, 0)),
            pl.BlockSpec((MEM, HIDDEN), full),
            pl.BlockSpec((MEM, HIDDEN), full),
            pl.BlockSpec((1, HIDDEN), full),
            pl.BlockSpec((1, HIDDEN), full),
            pl.BlockSpec((1, HIDDEN), full),
            pl.BlockSpec((1, HIDDEN), full),
            pl.BlockSpec((1, HIDDEN), full),
        ],
        out_specs=pl.BlockSpec((bn, HIDDEN), lambda i: (i, 0)),
        out_shape=jax.ShapeDtypeStruct((N, HIDDEN), jnp.float32),
    )(hidden, memory, Wk.T, Wv.T, vec(qn_w), vec(kn_w), vec(vn_w),
      vec(conv_w[:, 0, 2]), vec(conv_b))
    return out


# speedup vs baseline: 1.2193x; 1.0863x over previous
import functools
import math

import jax
import jax.numpy as jnp
from jax import lax
from jax.experimental import pallas as pl
from jax.experimental.pallas import tpu as pltpu
from jax.experimental.pallas import tpu_sc as plsc

HIDDEN = 1024
MEM = 512
BUCKETS = 100000
SLOTS = 8
SLOT_DIM = MEM // SLOTS
N = 16384

NC = 2
NS = 16
NW = NC * NS
ROWS_PER_W = N // NW
CHUNK = 128
CHUNKS_PER_SLOT = ROWS_PER_W // CHUNK


def _sc_gather(table_hbm, idx_hbm, out_hbm, idx_v, rows_v, sem):
    wid = lax.axis_index("s") * NC + lax.axis_index("c")
    n0 = wid * ROWS_PER_W
    pltpu.sync_copy(idx_hbm.at[:, pl.ds(n0, ROWS_PER_W)], idx_v)

    def body(j, carry):
        s = j // CHUNKS_PER_SLOT
        k = j % CHUNKS_PER_SLOT
        pltpu.async_copy(
            table_hbm.at[s].at[idx_v.at[s, pl.ds(k * CHUNK, CHUNK)]],
            rows_v, sem).wait()
        pltpu.sync_copy(
            rows_v, out_hbm.at[s, pl.ds(n0 + k * CHUNK, CHUNK), :])
        return carry

    lax.fori_loop(0, SLOTS * CHUNKS_PER_SLOT, body, 0)


def _make_gather_call():
    return functools.partial(
        pl.kernel,
        out_type=jax.ShapeDtypeStruct((SLOTS, N, 2 * SLOT_DIM), jnp.float32),
        mesh=plsc.VectorSubcoreMesh(core_axis_name="c", subcore_axis_name="s",
                                    num_cores=NC, num_subcores=NS),
        scratch_types=[
            pltpu.VMEM((SLOTS, ROWS_PER_W), jnp.int32),
            pltpu.VMEM((CHUNK, 2 * SLOT_DIM), jnp.float32),
            pltpu.SemaphoreType.DMA,
        ],
        compiler_params=pltpu.CompilerParams(use_tc_tiling_on_sc=True),
    )(_sc_gather)


def _dense_body(hid_ref, mem_ref, wkt_ref, wvt_ref, qn_ref, kn_ref, vn_ref,
                cw_ref, cb_ref, out_ref):
    eps = 1e-8
    q = hid_ref[...]
    q = q * lax.rsqrt(jnp.mean(q * q, axis=-1, keepdims=True) + eps)
    q = q * qn_ref[...]
    x = mem_ref[...]
    m = jnp.concatenate([x[s, :, :SLOT_DIM] for s in range(SLOTS)], axis=-1)
    m_hi = m.astype(jnp.bfloat16)

    def matmul(w):
        return jnp.dot(m_hi, w.astype(jnp.bfloat16),
                       preferred_element_type=jnp.float32)

    k = matmul(wkt_ref[...])
    k = k * lax.rsqrt(jnp.mean(k * k, axis=-1, keepdims=True) + eps)
    k = k * kn_ref[...]
    v = matmul(wvt_ref[...])
    v = v * lax.rsqrt(jnp.mean(v * v, axis=-1, keepdims=True) + eps)
    v = v * vn_ref[...]
    logits = jnp.sum(q * k, axis=-1, keepdims=True) * (1.0 / math.sqrt(HIDDEN))
    alpha = jax.nn.sigmoid(logits)
    g = alpha * v
    co = g * cw_ref[...] + cb_ref[...]
    out_ref[...] = co * jax.nn.sigmoid(co) + g


def kernel(hidden, batch_ngram_bucket_ids, tables, Wk, Wv, qn_w, kn_w, vn_w,
           conv_w, conv_b):
    idx = jnp.asarray(batch_ngram_bucket_ids, jnp.int32).T
    tab128 = jnp.pad(tables, ((0, 0), (0, 0), (0, SLOT_DIM)))

    rows = _make_gather_call()(tab128, idx)

    bn = 1024
    grid = (N // bn,)
    full = lambda i: (0, 0)
    vec = lambda x: x.reshape(1, HIDDEN)
    out = pl.pallas_call(
        _dense_body,
        grid=grid,
        in_specs=[
            pl.BlockSpec((bn, HIDDEN), lambda i: (i, 0)),
            pl.BlockSpec((SLOTS, bn, 2 * SLOT_DIM), lambda i: (0, i, 0)),
            pl.BlockSpec((MEM, HIDDEN), full),
            pl.BlockSpec((MEM, HIDDEN), full),
            pl.BlockSpec((1, HIDDEN), full),
            pl.BlockSpec((1, HIDDEN), full),
            pl.BlockSpec((1, HIDDEN), full),
            pl.BlockSpec((1, HIDDEN), full),
            pl.BlockSpec((1, HIDDEN), full),
        ],
        out_specs=pl.BlockSpec((bn, HIDDEN), lambda i: (i, 0)),
        out_shape=jax.ShapeDtypeStruct((N, HIDDEN), jnp.float32),
    )(hidden, rows, Wk.T, Wv.T, vec(qn_w), vec(kn_w), vec(vn_w),
      vec(conv_w[:, 0, 2]), vec(conv_b))
    return out
